# Initial kernel scaffold; baseline (speedup 1.0000x reference)
import jax
import jax.numpy as jnp
from jax.experimental import pallas as pl


def kernel(x, edge_index, W_gat, att_src, att_dst, bias_gat, bn_gamma, bn_beta, bn_mean, bn_var, W_lin, b_lin):
    raise NotImplementedError("write your pallas kernel here")


# speedup vs baseline: 14.5245x; 14.5245x over previous
import functools

import jax
import jax.numpy as jnp
from jax import lax
from jax.experimental import pallas as pl
from jax.experimental.pallas import tpu as pltpu
from jax.experimental.pallas import tpu_sc as plsc

N_NODES = 10000
N_EDGES = 320000
E1 = N_EDGES + N_NODES
IN_F = 128
OUT_F = 128
HEADS = 4

NP = 10240
EP = 331776
CHUNK = 128
TILES = 16
CHUNKS_PER_TILE = EP // (TILES * CHUNK)
ROW_CHUNKS_PER_TILE = NP // (TILES * CHUNK)

BLK = 1024
GRID = NP // BLK


def _pre_body(x_ref, w_ref, a8_ref, h0_ref, h1_ref, h2_ref, h3_ref, a_ref):
    xb = x_ref[...]
    outs = (h0_ref, h1_ref, h2_ref, h3_ref)
    hs = []
    for hh in range(HEADS):
        wh = w_ref[pl.ds(hh * OUT_F, OUT_F), :]
        hb = lax.dot_general(xb, wh, (((1,), (1,)), ((), ())),
                             preferred_element_type=jnp.float32)
        outs[hh][...] = hb
        hs.append(hb)
    hcat = jnp.concatenate(hs, axis=1)
    a_ref[...] = lax.dot_general(hcat, a8_ref[...], (((1,), (0,)), ((), ())),
                                 preferred_element_type=jnp.float32)


def _tc_pre(xp, w_gat, a8):
    out_shapes = tuple(jax.ShapeDtypeStruct((NP, OUT_F), jnp.float32)
                       for _ in range(HEADS))
    out_shapes = out_shapes + (jax.ShapeDtypeStruct((NP, 2 * HEADS), jnp.float32),)
    return pl.pallas_call(
        _pre_body,
        grid=(GRID,),
        in_specs=[
            pl.BlockSpec((BLK, IN_F), lambda i: (i, 0)),
            pl.BlockSpec((HEADS * OUT_F, IN_F), lambda i: (0, 0)),
            pl.BlockSpec((HEADS * OUT_F, 2 * HEADS), lambda i: (0, 0)),
        ],
        out_specs=[pl.BlockSpec((BLK, OUT_F), lambda i: (i, 0))] * HEADS
                  + [pl.BlockSpec((BLK, 2 * HEADS), lambda i: (i, 0))],
        out_shape=out_shapes,
    )(xp, w_gat, a8)


def _post_body(h0_ref, h1_ref, h2_ref, h3_ref, bias_ref, gamma_ref, beta_ref,
               mean_ref, var_ref, wl_ref, bl_ref, o_ref):
    z = jnp.concatenate([h0_ref[...], h1_ref[...], h2_ref[...], h3_ref[...]],
                        axis=1)
    s = gamma_ref[...] * lax.rsqrt(var_ref[...] + 1e-5)
    t0 = (bias_ref[...] - mean_ref[...]) * s + beta_ref[...]
    y = jnp.maximum(z * s[None, :] + t0[None, :], 0.0)
    r = lax.dot_general(y, wl_ref[...], (((1,), (1,)), ((), ())),
                        preferred_element_type=jnp.float32)
    o_ref[...] = r + bl_ref[...][None, :]


def _tc_post(h0, h1, h2, h3, bias, gamma, beta, mean, var, w_lin, b_lin):
    d = HEADS * OUT_F
    vec = lambda: pl.BlockSpec((d,), lambda i: (0,))
    return pl.pallas_call(
        _post_body,
        grid=(GRID,),
        in_specs=[pl.BlockSpec((BLK, OUT_F), lambda i: (i, 0))] * HEADS + [
            vec(), vec(), vec(), vec(), vec(),
            pl.BlockSpec((OUT_F, d), lambda i: (0, 0)),
            pl.BlockSpec((OUT_F,), lambda i: (0,)),
        ],
        out_specs=pl.BlockSpec((BLK, OUT_F), lambda i: (i, 0)),
        out_shape=jax.ShapeDtypeStruct((NP, OUT_F), jnp.float32),
    )(h0, h1, h2, h3, bias, gamma, beta, mean, var, w_lin, b_lin)


_GDN = lax.GatherDimensionNumbers(offset_dims=(), collapsed_slice_dims=(0,),
                                  start_index_map=(0,))


def _bcast_lane(v, j):
    idx = jnp.full((16, 1), j, jnp.int32)
    return lax.gather(v, idx, _GDN, (1,),
                      mode=lax.GatherScatterMode.PROMISE_IN_BOUNDS)

def _sc_body(src_hbm, dst_hbm, as0, as1, as2, as3, ad0, ad1, ad2, ad3,
             h0_hbm, h1_hbm, h2_hbm, h3_hbm, o0_hbm, o1_hbm, o2_hbm, o3_hbm,
             rows_v, av_v, bv_v, srcv, dstv, wv_v, acc_s, den_s, sem):
    cid = lax.axis_index("c")
    sid = lax.axis_index("s")
    h_tabs = (h0_hbm, h1_hbm, h2_hbm, h3_hbm)
    o_tabs = (o0_hbm, o1_hbm, o2_hbm, o3_hbm)
    as_tabs = (as0, as1, as2, as3)
    ad_tabs = (ad0, ad1, ad2, ad3)
    zero16 = jnp.zeros((16,), jnp.float32)

    def zero_rows_buf(_i, _):
        for g in range(8):
            rows_v[_i, pl.ds(g * 16, 16)] = zero16
        return _

    for hh in range(HEADS):
        @pl.when(cid == hh // 2)
        def _pass(hh=hh):
            h_hbm = h_tabs[hh]
            o_hbm = o_tabs[hh]
            asrc_hbm = as_tabs[hh]
            adst_hbm = ad_tabs[hh]

            lax.fori_loop(0, CHUNK, zero_rows_buf, None)
            for g in range(8):
                wv_v[pl.ds(g * 16, 16)] = zero16
            for j in range(ROW_CHUNKS_PER_TILE):
                r0 = (sid * ROW_CHUNKS_PER_TILE + j) * CHUNK
                pltpu.sync_copy(rows_v, acc_s.at[pl.ds(r0, CHUNK)])
                pltpu.sync_copy(wv_v, den_s.at[pl.ds(r0, CHUNK)])
            plsc.subcore_barrier()

            def chunk_body(i, _):
                base = (sid * CHUNKS_PER_TILE + i) * CHUNK
                pltpu.sync_copy(src_hbm.at[pl.ds(base, CHUNK)], srcv)
                pltpu.sync_copy(dst_hbm.at[pl.ds(base, CHUNK)], dstv)
                pltpu.async_copy(asrc_hbm.at[srcv], av_v, sem).wait()
                pltpu.async_copy(adst_hbm.at[dstv], bv_v, sem).wait()
                pltpu.async_copy(h_hbm.at[srcv], rows_v, sem).wait()
                for g in range(8):
                    l = av_v[pl.ds(g * 16, 16)] + bv_v[pl.ds(g * 16, 16)]
                    m = jnp.maximum(l, l * jnp.float32(0.2))
                    wv_v[pl.ds(g * 16, 16)] = jnp.exp(m)

                def scale_grp(kg, _):
                    wgrp = wv_v[pl.ds(kg * 16, 16)]
                    for j in range(16):
                        wb = _bcast_lane(wgrp, j)
                        k = kg * 16 + j
                        for g in range(8):
                            rows_v[k, pl.ds(g * 16, 16)] = (
                                rows_v[k, pl.ds(g * 16, 16)] * wb)
                    return _
                lax.fori_loop(0, CHUNK // 16, scale_grp, None)

                pltpu.sync_copy(wv_v, den_s.at[dstv], add=True)
                pltpu.sync_copy(rows_v, acc_s.at[dstv], add=True)
                return _
            lax.fori_loop(0, CHUNKS_PER_TILE, chunk_body, None)
            plsc.subcore_barrier()

            for j in range(ROW_CHUNKS_PER_TILE):
                r0 = (sid * ROW_CHUNKS_PER_TILE + j) * CHUNK
                pltpu.sync_copy(acc_s.at[pl.ds(r0, CHUNK)], rows_v)
                pltpu.sync_copy(den_s.at[pl.ds(r0, CHUNK)], wv_v)

                def norm_grp(kg, _):
                    dgrp = wv_v[pl.ds(kg * 16, 16)]
                    invg = jnp.float32(1.0) / (dgrp + jnp.float32(1e-16))
                    for j in range(16):
                        ib = _bcast_lane(invg, j)
                        k = kg * 16 + j
                        for g in range(8):
                            rows_v[k, pl.ds(g * 16, 16)] = (
                                rows_v[k, pl.ds(g * 16, 16)] * ib)
                    return _
                lax.fori_loop(0, CHUNK // 16, norm_grp, None)
                pltpu.sync_copy(rows_v, o_hbm.at[pl.ds(r0, CHUNK)])
            plsc.subcore_barrier()


def _sc_edge(src_i, dst_i, as_cols, ad_cols, h0, h1, h2, h3):
    mesh = plsc.VectorSubcoreMesh(core_axis_name="c", subcore_axis_name="s",
                                  num_cores=2, num_subcores=TILES)
    fn = pl.kernel(
        _sc_body,
        out_type=tuple(jax.ShapeDtypeStruct((NP, OUT_F), jnp.float32)
                       for _ in range(HEADS)),
        mesh=mesh,
        scratch_types=[
            pltpu.VMEM((CHUNK, OUT_F), jnp.float32),
            pltpu.VMEM((CHUNK,), jnp.float32),
            pltpu.VMEM((CHUNK,), jnp.float32),
            pltpu.VMEM((CHUNK,), jnp.int32),
            pltpu.VMEM((CHUNK,), jnp.int32),
            pltpu.VMEM((CHUNK,), jnp.float32),
            pltpu.VMEM_SHARED((NP, OUT_F), jnp.float32),
            pltpu.VMEM_SHARED((NP,), jnp.float32),
            pltpu.SemaphoreType.DMA,
        ],
    )
    return fn(src_i, dst_i, *as_cols, *ad_cols, h0, h1, h2, h3)


@jax.jit
def kernel(x, edge_index, W_gat, att_src, att_dst, bias_gat, bn_gamma,
           bn_beta, bn_mean, bn_var, W_lin, b_lin):
    n = x.shape[0]
    xp = jnp.pad(x, ((0, NP - n), (0, 0)))
    loop = jnp.arange(n, dtype=jnp.int32)
    src = jnp.concatenate([edge_index[0].astype(jnp.int32), loop])
    dst = jnp.concatenate([edge_index[1].astype(jnp.int32), loop])
    src = jnp.pad(src, (0, EP - E1))
    dst = jnp.pad(dst, (0, EP - E1), constant_values=n)

    att_s = att_src[0]
    att_d = att_dst[0]
    eye = jnp.eye(HEADS, dtype=jnp.float32)
    a8 = jnp.concatenate([
        (att_s[:, :, None] * eye[:, None, :]).reshape(HEADS * OUT_F, HEADS),
        (att_d[:, :, None] * eye[:, None, :]).reshape(HEADS * OUT_F, HEADS),
    ], axis=1)

    h0, h1, h2, h3, a_all = _tc_pre(xp, W_gat, a8)
    as_cols = [a_all[:, hh] + 0.0 for hh in range(HEADS)]
    ad_cols = [a_all[:, HEADS + hh] + 0.0 for hh in range(HEADS)]

    o0, o1, o2, o3 = _sc_edge(src, dst, as_cols, ad_cols, h0, h1, h2, h3)

    out = _tc_post(o0, o1, o2, o3, bias_gat, bn_gamma, bn_beta, bn_mean,
                   bn_var, W_lin, b_lin)
    return out[:n]


# speedup vs baseline: 25.6529x; 1.7662x over previous
import functools

import jax
import jax.numpy as jnp
from jax import lax
from jax.experimental import pallas as pl
from jax.experimental.pallas import tpu as pltpu
from jax.experimental.pallas import tpu_sc as plsc

N_NODES = 10000
N_EDGES = 320000
E1 = N_EDGES + N_NODES
IN_F = 128
OUT_F = 128
HEADS = 4

NP = 10240
EP = 331776
CHUNK = 128
TILES = 16
CHUNKS_PER_TILE = EP // (TILES * CHUNK)
ROW_CHUNKS_PER_TILE = NP // (TILES * CHUNK)

BLK = 1024
GRID = NP // BLK


def _pre_body(x_ref, w_ref, a8_ref, h0_ref, h1_ref, h2_ref, h3_ref, a_ref):
    xb = x_ref[...]
    outs = (h0_ref, h1_ref, h2_ref, h3_ref)
    hs = []
    for hh in range(HEADS):
        wh = w_ref[pl.ds(hh * OUT_F, OUT_F), :]
        hb = lax.dot_general(xb, wh, (((1,), (1,)), ((), ())),
                             preferred_element_type=jnp.float32)
        outs[hh][...] = hb
        hs.append(hb)
    hcat = jnp.concatenate(hs, axis=1)
    a_ref[...] = lax.dot_general(hcat, a8_ref[...], (((1,), (0,)), ((), ())),
                                 preferred_element_type=jnp.float32)


def _tc_pre(xp, w_gat, a8):
    out_shapes = tuple(jax.ShapeDtypeStruct((NP, OUT_F), jnp.float32)
                       for _ in range(HEADS))
    out_shapes = out_shapes + (jax.ShapeDtypeStruct((NP, 2 * HEADS), jnp.float32),)
    return pl.pallas_call(
        _pre_body,
        grid=(GRID,),
        in_specs=[
            pl.BlockSpec((BLK, IN_F), lambda i: (i, 0)),
            pl.BlockSpec((HEADS * OUT_F, IN_F), lambda i: (0, 0)),
            pl.BlockSpec((HEADS * OUT_F, 2 * HEADS), lambda i: (0, 0)),
        ],
        out_specs=[pl.BlockSpec((BLK, OUT_F), lambda i: (i, 0))] * HEADS
                  + [pl.BlockSpec((BLK, 2 * HEADS), lambda i: (i, 0))],
        out_shape=out_shapes,
    )(xp, w_gat, a8)


def _post_body(h0_ref, h1_ref, h2_ref, h3_ref, bias_ref, gamma_ref, beta_ref,
               mean_ref, var_ref, wl_ref, bl_ref, o_ref):
    z = jnp.concatenate([h0_ref[...], h1_ref[...], h2_ref[...], h3_ref[...]],
                        axis=1)
    s = gamma_ref[...] * lax.rsqrt(var_ref[...] + 1e-5)
    t0 = (bias_ref[...] - mean_ref[...]) * s + beta_ref[...]
    y = jnp.maximum(z * s[None, :] + t0[None, :], 0.0)
    r = lax.dot_general(y, wl_ref[...], (((1,), (1,)), ((), ())),
                        preferred_element_type=jnp.float32)
    o_ref[...] = r + bl_ref[...][None, :]


def _tc_post(h0, h1, h2, h3, bias, gamma, beta, mean, var, w_lin, b_lin):
    d = HEADS * OUT_F
    vec = lambda: pl.BlockSpec((d,), lambda i: (0,))
    return pl.pallas_call(
        _post_body,
        grid=(GRID,),
        in_specs=[pl.BlockSpec((BLK, OUT_F), lambda i: (i, 0))] * HEADS + [
            vec(), vec(), vec(), vec(), vec(),
            pl.BlockSpec((OUT_F, d), lambda i: (0, 0)),
            pl.BlockSpec((OUT_F,), lambda i: (0,)),
        ],
        out_specs=pl.BlockSpec((BLK, OUT_F), lambda i: (i, 0)),
        out_shape=jax.ShapeDtypeStruct((NP, OUT_F), jnp.float32),
    )(h0, h1, h2, h3, bias, gamma, beta, mean, var, w_lin, b_lin)


_GDN = lax.GatherDimensionNumbers(offset_dims=(), collapsed_slice_dims=(0,),
                                  start_index_map=(0,))


def _bcast_lane(v, j):
    idx = jnp.full((16, 1), j, jnp.int32)
    return lax.gather(v, idx, _GDN, (1,),
                      mode=lax.GatherScatterMode.PROMISE_IN_BOUNDS)

def _sc_body(src_hbm, dst_hbm, as0, as1, as2, as3, ad0, ad1, ad2, ad3,
             h0_hbm, h1_hbm, h2_hbm, h3_hbm, o0_hbm, o1_hbm, o2_hbm, o3_hbm,
             rows_v, av_v, bv_v, srcb, dstb, dstw, wv_v, acc_s, den_s, sem):
    cid = lax.axis_index("c")
    sid = lax.axis_index("s")
    h_tabs = (h0_hbm, h1_hbm, h2_hbm, h3_hbm)
    o_tabs = (o0_hbm, o1_hbm, o2_hbm, o3_hbm)
    as_tabs = (as0, as1, as2, as3)
    ad_tabs = (ad0, ad1, ad2, ad3)
    zero16 = jnp.zeros((16,), jnp.float32)

    def zero_rows_buf(_i, _):
        for g in range(8):
            rows_v[_i, pl.ds(g * 16, 16)] = zero16
        return _

    for hh in range(HEADS):
        @pl.when(cid == hh // 2)
        def _pass(hh=hh):
            h_hbm = h_tabs[hh]
            o_hbm = o_tabs[hh]
            asrc_hbm = as_tabs[hh]
            adst_hbm = ad_tabs[hh]

            lax.fori_loop(0, CHUNK, zero_rows_buf, None)
            for g in range(8):
                wv_v[pl.ds(g * 16, 16)] = zero16
            for j in range(ROW_CHUNKS_PER_TILE):
                r0 = (sid * ROW_CHUNKS_PER_TILE + j) * CHUNK
                pltpu.sync_copy(rows_v, acc_s.at[pl.ds(r0, CHUNK)])
                pltpu.sync_copy(wv_v, den_s.at[pl.ds(r0, CHUNK)])
            plsc.subcore_barrier()

            def chunk_body(i, _):
                srcv = srcb.at[pl.ds(i * CHUNK, CHUNK)]
                dstv = dstb.at[pl.ds(i * CHUNK, CHUNK)]
                for g in range(8):
                    dstw[pl.ds(g * 16, 16)] = dstb[pl.ds(i * CHUNK + g * 16,
                                                         16)]
                d1 = pltpu.async_copy(asrc_hbm.at[srcv], av_v, sem)
                d2 = pltpu.async_copy(adst_hbm.at[dstv], bv_v, sem)
                d3 = pltpu.async_copy(h_hbm.at[srcv], rows_v, sem)
                d1.wait()
                d2.wait()
                d3.wait()
                for g in range(8):
                    l = av_v[pl.ds(g * 16, 16)] + bv_v[pl.ds(g * 16, 16)]
                    m = jnp.maximum(l, l * jnp.float32(0.2))
                    wv_v[pl.ds(g * 16, 16)] = jnp.exp(m)

                def scale_grp(kg, _):
                    wgrp = wv_v[pl.ds(kg * 16, 16)]
                    for j in range(16):
                        wb = _bcast_lane(wgrp, j)
                        k = kg * 16 + j
                        for g in range(8):
                            rows_v[k, pl.ds(g * 16, 16)] = (
                                rows_v[k, pl.ds(g * 16, 16)] * wb)
                    return _
                lax.fori_loop(0, CHUNK // 16, scale_grp, None)

                pltpu.sync_copy(wv_v, den_s.at[dstw], add=True)
                pltpu.sync_copy(rows_v, acc_s.at[dstw], add=True)
                return _

            half = CHUNKS_PER_TILE // 2 * CHUNK
            for hb in range(2):
                lo = (sid * CHUNKS_PER_TILE) * CHUNK + hb * half
                pltpu.sync_copy(src_hbm.at[pl.ds(lo, half)], srcb)
                pltpu.sync_copy(dst_hbm.at[pl.ds(lo, half)], dstb)
                lax.fori_loop(0, CHUNKS_PER_TILE // 2, chunk_body, None)
            plsc.subcore_barrier()

            for j in range(ROW_CHUNKS_PER_TILE):
                r0 = (sid * ROW_CHUNKS_PER_TILE + j) * CHUNK
                pltpu.sync_copy(acc_s.at[pl.ds(r0, CHUNK)], rows_v)
                pltpu.sync_copy(den_s.at[pl.ds(r0, CHUNK)], wv_v)

                def norm_grp(kg, _):
                    dgrp = wv_v[pl.ds(kg * 16, 16)]
                    invg = jnp.float32(1.0) / (dgrp + jnp.float32(1e-16))
                    for j in range(16):
                        ib = _bcast_lane(invg, j)
                        k = kg * 16 + j
                        for g in range(8):
                            rows_v[k, pl.ds(g * 16, 16)] = (
                                rows_v[k, pl.ds(g * 16, 16)] * ib)
                    return _
                lax.fori_loop(0, CHUNK // 16, norm_grp, None)
                pltpu.sync_copy(rows_v, o_hbm.at[pl.ds(r0, CHUNK)])
            plsc.subcore_barrier()


def _sc_edge(src_i, dst_i, as_cols, ad_cols, h0, h1, h2, h3):
    mesh = plsc.VectorSubcoreMesh(core_axis_name="c", subcore_axis_name="s",
                                  num_cores=2, num_subcores=TILES)
    fn = pl.kernel(
        _sc_body,
        out_type=tuple(jax.ShapeDtypeStruct((NP, OUT_F), jnp.float32)
                       for _ in range(HEADS)),
        mesh=mesh,
        scratch_types=[
            pltpu.VMEM((CHUNK, OUT_F), jnp.float32),
            pltpu.VMEM((CHUNK,), jnp.float32),
            pltpu.VMEM((CHUNK,), jnp.float32),
            pltpu.VMEM((CHUNKS_PER_TILE // 2 * CHUNK,), jnp.int32),
            pltpu.VMEM((CHUNKS_PER_TILE // 2 * CHUNK,), jnp.int32),
            pltpu.VMEM((CHUNK,), jnp.int32),
            pltpu.VMEM((CHUNK,), jnp.float32),
            pltpu.VMEM_SHARED((NP, OUT_F), jnp.float32),
            pltpu.VMEM_SHARED((NP,), jnp.float32),
            pltpu.SemaphoreType.DMA,
        ],
    )
    return fn(src_i, dst_i, *as_cols, *ad_cols, h0, h1, h2, h3)


@jax.jit
def kernel(x, edge_index, W_gat, att_src, att_dst, bias_gat, bn_gamma,
           bn_beta, bn_mean, bn_var, W_lin, b_lin):
    n = x.shape[0]
    xp = jnp.pad(x, ((0, NP - n), (0, 0)))
    loop = jnp.arange(n, dtype=jnp.int32)
    padi = jnp.arange(EP - E1, dtype=jnp.int32)
    src = jnp.concatenate([edge_index[0].astype(jnp.int32), loop,
                           padi % n])
    dst = jnp.concatenate([edge_index[1].astype(jnp.int32), loop,
                           n + padi % (NP - n)])

    att_s = att_src[0]
    att_d = att_dst[0]
    eye = jnp.eye(HEADS, dtype=jnp.float32)
    a8 = jnp.concatenate([
        (att_s[:, :, None] * eye[:, None, :]).reshape(HEADS * OUT_F, HEADS),
        (att_d[:, :, None] * eye[:, None, :]).reshape(HEADS * OUT_F, HEADS),
    ], axis=1)

    h0, h1, h2, h3, a_all = _tc_pre(xp, W_gat, a8)
    as_cols = [a_all[:, hh] + 0.0 for hh in range(HEADS)]
    ad_cols = [a_all[:, HEADS + hh] + 0.0 for hh in range(HEADS)]

    o0, o1, o2, o3 = _sc_edge(src, dst, as_cols, ad_cols, h0, h1, h2, h3)

    out = _tc_post(o0, o1, o2, o3, bias_gat, bn_gamma, bn_beta, bn_mean,
                   bn_var, W_lin, b_lin)
    return out[:n]
